# baseline (speedup 1.0000x reference)
import functools

import jax
import jax.numpy as jnp
from jax import lax
from jax.experimental import pallas as pl
from jax.experimental.pallas import tpu as pltpu
from jax.experimental.pallas import tpu_sc as plsc

SHAPE = (8, 32, 16, 64, 128)
ROW = 128
N = 8 * 32 * 16 * 64 * 128
NUM_WORKERS = 32
WORDS_PER_WORKER = N // NUM_WORKERS
CHUNK = 16384
NUM_CHUNKS = WORDS_PER_WORKER // CHUNK
ROWS_PER_CHUNK = CHUNK // ROW
GROUPS_PER_ROW = ROW // 16


def _body(x_hbm, out_hbm, in0, in1, out0, out1, si0, si1, so0, so1):
    wid = lax.axis_index("s") * 2 + lax.axis_index("c")
    base0 = wid * WORDS_PER_WORKER
    iota = lax.iota(jnp.int32, 16)
    pat0 = jnp.where(iota == 0, 1, iota - 1)
    pats = [pat0] + [16 * g - 1 + iota for g in range(1, GROUPS_PER_ROW)]
    in_bufs, out_bufs = (in0, in1), (out0, out1)
    in_sems, out_sems = (si0, si1), (so0, so1)

    def start_in(c, b):
        src = x_hbm.at[pl.ds(base0 + c * CHUNK, CHUNK)]
        pltpu.make_async_copy(src, in_bufs[b], in_sems[b]).start()

    def wait_in(b):
        pltpu.make_async_copy(
            x_hbm.at[pl.ds(0, CHUNK)], in_bufs[b], in_sems[b]).wait()

    def start_out(c, b):
        dst = out_hbm.at[pl.ds(base0 + c * CHUNK, CHUNK)]
        pltpu.make_async_copy(out_bufs[b], dst, out_sems[b]).start()

    def wait_out(b):
        pltpu.make_async_copy(
            out_bufs[b], out_hbm.at[pl.ds(0, CHUNK)], out_sems[b]).wait()

    def compute(b):
        ib, ob = in_bufs[b], out_bufs[b]

        def row(i, carry):
            base = i * ROW
            for g in range(GROUPS_PER_ROW):
                w = plsc.load_gather(ib, [base + pats[g]])
                ob[pl.ds(base + 16 * g, 16)] = w
            return carry

        lax.fori_loop(0, ROWS_PER_CHUNK, row, 0, unroll=8)

    start_in(0, 0)
    start_in(1, 1)

    def step(g, carry):
        for b in range(2):
            c = 2 * g + b
            wait_in(b)

            @pl.when(c >= 2)
            def _():
                wait_out(b)

            compute(b)
            start_out(c, b)

            @pl.when(c + 2 < NUM_CHUNKS)
            def _():
                start_in(c + 2, b)
        return carry

    lax.fori_loop(0, NUM_CHUNKS // 2, step, 0)
    wait_out(0)
    wait_out(1)


@jax.jit
def kernel(x):
    xf = x.reshape(N)
    mesh = plsc.VectorSubcoreMesh(core_axis_name="c", subcore_axis_name="s")
    out = pl.kernel(
        _body,
        mesh=mesh,
        out_type=jax.ShapeDtypeStruct((N,), jnp.float32),
        scratch_types=[pltpu.VMEM((CHUNK,), jnp.float32)] * 4
        + [pltpu.SemaphoreType.DMA] * 4,
        compiler_params=pltpu.CompilerParams(needs_layout_passes=False),
    )(xf)
    return out.reshape(SHAPE)


# speedup vs baseline: 1.0678x; 1.0678x over previous
import functools

import jax
import jax.numpy as jnp
from jax import lax
from jax.experimental import pallas as pl
from jax.experimental.pallas import tpu as pltpu
from jax.experimental.pallas import tpu_sc as plsc

SHAPE = (8, 32, 16, 64, 128)
ROW = 128
N = 8 * 32 * 16 * 64 * 128
NUM_WORKERS = 32
WORDS_PER_WORKER = N // NUM_WORKERS
CHUNK = 16384
NUM_CHUNKS = WORDS_PER_WORKER // CHUNK
ROWS_PER_CHUNK = CHUNK // ROW
GROUPS_PER_ROW = ROW // 16


def _body(x_hbm, out_hbm, in0, in1, out0, out1, si0, si1, so0, so1):
    wid = lax.axis_index("s") * 2 + lax.axis_index("c")
    base0 = wid * WORDS_PER_WORKER
    iota = lax.iota(jnp.int32, 16)
    in_bufs, out_bufs = (in0, in1), (out0, out1)
    in_sems, out_sems = (si0, si1), (so0, so1)

    def start_in(c, b):
        src = x_hbm.at[pl.ds(base0 + c * CHUNK, CHUNK)]
        pltpu.make_async_copy(src, in_bufs[b], in_sems[b]).start()

    def wait_in(b):
        pltpu.make_async_copy(
            x_hbm.at[pl.ds(0, CHUNK)], in_bufs[b], in_sems[b]).wait()

    def start_out(c, b):
        dst = out_hbm.at[pl.ds(base0 + c * CHUNK, CHUNK)]
        src = out_bufs[b].at[pl.ds(0, CHUNK)]
        pltpu.make_async_copy(src, dst, out_sems[b]).start()

    def wait_out(b):
        src = out_bufs[b].at[pl.ds(0, CHUNK)]
        pltpu.make_async_copy(
            src, out_hbm.at[pl.ds(0, CHUNK)], out_sems[b]).wait()

    def compute(b):
        ib, ob = in_bufs[b], out_bufs[b]

        def grp(i, carry):
            base = i * 16
            w = ib[pl.ds(base, 16)]
            ob[pl.ds(base + 1, 16)] = w
            return carry

        lax.fori_loop(0, CHUNK // 16, grp, 0, unroll=8)

        def fix(r, carry):
            pos = (r * 16 + iota) * ROW
            w = plsc.load_gather(ib, [pos + 1])
            plsc.store_scatter(ob, [pos], w)
            return carry

        lax.fori_loop(0, ROWS_PER_CHUNK // 16, fix, 0, unroll=2)

    start_in(0, 0)
    start_in(1, 1)

    def step(g, carry):
        for b in range(2):
            c = 2 * g + b
            wait_in(b)

            @pl.when(c >= 2)
            def _():
                wait_out(b)

            compute(b)
            start_out(c, b)

            @pl.when(c + 2 < NUM_CHUNKS)
            def _():
                start_in(c + 2, b)
        return carry

    lax.fori_loop(0, NUM_CHUNKS // 2, step, 0)
    wait_out(0)
    wait_out(1)


@jax.jit
def kernel(x):
    xf = x.reshape(N)
    mesh = plsc.VectorSubcoreMesh(core_axis_name="c", subcore_axis_name="s")
    out = pl.kernel(
        _body,
        mesh=mesh,
        out_type=jax.ShapeDtypeStruct((N,), jnp.float32),
        scratch_types=[pltpu.VMEM((CHUNK,), jnp.float32)] * 2
        + [pltpu.VMEM((CHUNK + 8,), jnp.float32)] * 2
        + [pltpu.SemaphoreType.DMA] * 4,
        compiler_params=pltpu.CompilerParams(needs_layout_passes=False),
    )(xf)
    return out.reshape(SHAPE)


# speedup vs baseline: 2.3679x; 2.2175x over previous
import jax
import jax.numpy as jnp
from jax import lax
from jax.experimental import pallas as pl
from jax.experimental.pallas import tpu as pltpu
from jax.experimental.pallas import tpu_sc as plsc

SHAPE = (8, 32, 16, 64, 128)
ROW = 128
N = 8 * 32 * 16 * 64 * 128
NUM_WORKERS = 32
WORDS_PER_WORKER = N // NUM_WORKERS
CHUNK = 16384
NUM_CHUNKS = WORDS_PER_WORKER // CHUNK
ROWS_PER_CHUNK = CHUNK // ROW
GROUPS_PER_ROW = ROW // 16


def _body(x_hbm, out_hbm, in0, in1, in2, in3, out0, out1, out2,
          si0, si1, si2, si3, so0, so1, so2):
    wid = lax.axis_index("s") * 2 + lax.axis_index("c")
    base0 = wid * WORDS_PER_WORKER
    iota = lax.iota(jnp.int32, 16)
    in_bufs, out_bufs = (in0, in1, in2, in3), (out0, out1, out2)
    in_sems, out_sems = (si0, si1, si2, si3), (so0, so1, so2)

    def start_in(c, b):
        src = x_hbm.at[pl.ds(base0 + c * CHUNK, CHUNK)]
        pltpu.make_async_copy(src, in_bufs[b], in_sems[b]).start()

    def wait_in(b):
        pltpu.make_async_copy(
            x_hbm.at[pl.ds(0, CHUNK)], in_bufs[b], in_sems[b]).wait()

    HALF = CHUNK // 2

    def start_out_half(c, b, h):
        dst = out_hbm.at[pl.ds(base0 + c * CHUNK + h * HALF, HALF)]
        src = out_bufs[b].at[pl.ds(h * HALF, HALF)]
        pltpu.make_async_copy(src, dst, out_sems[b]).start()

    def wait_out(b):
        src = out_bufs[b].at[pl.ds(0, CHUNK)]
        pltpu.make_async_copy(
            src, out_hbm.at[pl.ds(0, CHUNK)], out_sems[b]).wait()

    def compute_full(bi, bo):
        ib, ob = in_bufs[bi], out_bufs[bo]
        r0, r1 = 0, ROWS_PER_CHUNK

        def row1(i, ws):
            base = i * ROW
            nxt = tuple(
                ib[pl.ds(base + 16 * k, 16)] for k in range(GROUPS_PER_ROW))
            for k in range(GROUPS_PER_ROW):
                ob[pl.ds(base - ROW + 16 * k + 1, 16)] = ws[k]
            return nxt

        ws0 = tuple(
            ib[pl.ds(r0 * ROW + 16 * k, 16)] for k in range(GROUPS_PER_ROW))
        wsl = lax.fori_loop(r0 + 1, r1, row1, ws0)
        last = (r1 - 1) * ROW
        for k in range(GROUPS_PER_ROW):
            ob[pl.ds(last + 16 * k + 1, 16)] = wsl[k]

        poss = [(r * 16 + iota) * ROW for r in range(r0 // 16, r1 // 16)]
        ws = [plsc.load_gather(ib, [pos + 1]) for pos in poss]
        for pos, w in zip(poss, ws):
            plsc.store_scatter(ob, [pos], w)

    start_in(0, 0)
    start_in(1, 1)
    start_in(2, 2)
    for c in range(NUM_CHUNKS):
        bi, bo = c % 4, c % 3
        wait_in(bi)
        if c >= 3:
            wait_out(bo)
        if c + 3 < NUM_CHUNKS:
            start_in(c + 3, (c + 3) % 4)
        compute_full(bi, bo)
        start_out_half(c, bo, 0)
        start_out_half(c, bo, 1)
    for b in range(3):
        if NUM_CHUNKS - 3 + b >= 0:
            wait_out((NUM_CHUNKS - 3 + b) % 3)


@jax.jit
def kernel(x):
    xf = x.reshape(N)
    mesh = plsc.VectorSubcoreMesh(core_axis_name="c", subcore_axis_name="s")
    out = pl.kernel(
        _body,
        mesh=mesh,
        out_type=jax.ShapeDtypeStruct((N,), jnp.float32),
        scratch_types=[pltpu.VMEM((CHUNK,), jnp.float32)] * 4
        + [pltpu.VMEM((CHUNK + 8,), jnp.float32)] * 3
        + [pltpu.SemaphoreType.DMA] * 7,
        compiler_params=pltpu.CompilerParams(needs_layout_passes=False),
    )(xf)
    return out.reshape(SHAPE)
